# Initial kernel scaffold; baseline (speedup 1.0000x reference)
import jax
import jax.numpy as jnp
from jax.experimental import pallas as pl


def kernel(positions, max_neighbours):
    raise NotImplementedError("write your pallas kernel here")


# speedup vs baseline: 75.4315x; 75.4315x over previous
import functools

import jax
import jax.numpy as jnp
import numpy as np
from jax import lax
from jax.experimental import pallas as pl
from jax.experimental.pallas import tpu as pltpu
from jax.experimental.pallas import tpu_sc as plsc

N_POINTS = 1024
N_CELLS = 27
MAXN = 64
LARGE = np.int32(1 << 30)
THRESH = np.float32(0.22 * 0.22)
NC = 2
NS = 16
NW = NC * NS
CPW = N_POINTS // NW


def _vsort(v):
    return jnp.sort(v)


def _rev(v):
    return jnp.flip(v, 0)


def _merge2(a, b):
    rb = _rev(b)
    return _vsort(jnp.minimum(a, rb)), _vsort(jnp.maximum(a, rb))


def _bmerge32(p, q):
    return _vsort(jnp.minimum(p, q)), _vsort(jnp.maximum(p, q))


def _sort64(a0, a1, a2, a3):
    a0, a1, a2, a3 = _vsort(a0), _vsort(a1), _vsort(a2), _vsort(a3)
    b0, b1 = _merge2(a0, a1)
    b2, b3 = _merge2(a2, a3)
    r3, r2 = _rev(b3), _rev(b2)
    l0, l1 = jnp.minimum(b0, r3), jnp.minimum(b1, r2)
    h0, h1 = jnp.maximum(b0, r3), jnp.maximum(b1, r2)
    s0, s1 = _bmerge32(l0, l1)
    s2, s3 = _bmerge32(h0, h1)
    return s0, s1, s2, s3


def _merge_low64(a, b):
    a0, a1, a2, a3 = a
    b0, b1, b2, b3 = b
    l0 = jnp.minimum(a0, _rev(b3))
    l1 = jnp.minimum(a1, _rev(b2))
    l2 = jnp.minimum(a2, _rev(b1))
    l3 = jnp.minimum(a3, _rev(b0))
    p0, p1 = jnp.minimum(l0, l2), jnp.minimum(l1, l3)
    q0, q1 = jnp.maximum(l0, l2), jnp.maximum(l1, l3)
    s0, s1 = _bmerge32(p0, p1)
    s2, s3 = _bmerge32(q0, q1)
    return s0, s1, s2, s3


@functools.partial(
    pl.kernel,
    mesh=plsc.VectorSubcoreMesh(core_axis_name="c", subcore_axis_name="s"),
    compiler_params=pltpu.CompilerParams(needs_layout_passes=False),
    out_type=[
        jax.ShapeDtypeStruct((N_POINTS, MAXN), jnp.int32),
        jax.ShapeDtypeStruct((N_POINTS,), jnp.int32),
    ],
    scratch_types=[
        pltpu.VMEM((N_POINTS,), jnp.float32),
        pltpu.VMEM((N_POINTS,), jnp.float32),
        pltpu.VMEM((N_POINTS,), jnp.float32),
        pltpu.VMEM((N_POINTS + MAXN,), jnp.int32),
        pltpu.VMEM((CPW, MAXN), jnp.int32),
        pltpu.VMEM((CPW,), jnp.int32),
    ],
)
def _neighbour_kernel(x_hbm, y_hbm, z_hbm, keys_hbm, counts_hbm,
                      xv, yv, zv, buf, kstage, cstage):
    wid = lax.axis_index("s") * NC + lax.axis_index("c")
    pltpu.sync_copy(x_hbm, xv)
    pltpu.sync_copy(y_hbm, yv)
    pltpu.sync_copy(z_hbm, zv)

    half = jnp.float32(0.5)
    one = jnp.float32(1.0)
    zero = jnp.float32(0.0)
    lane = lax.iota(jnp.int32, 16)
    largev = jnp.full((16,), LARGE, jnp.int32)

    cbase = wid * CPW
    xc = [xv[pl.ds(cbase, 16)], xv[pl.ds(cbase + 16, 16)]]
    yc = [yv[pl.ds(cbase, 16)], yv[pl.ds(cbase + 16, 16)]]
    zc = [zv[pl.ds(cbase, 16)], zv[pl.ds(cbase + 16, 16)]]

    def _splat(pair, sel, in_hi):
        w = jnp.where(in_hi, pair[1], pair[0])
        return jnp.max(jnp.where(sel, w, jnp.float32(-1.0)))

    def center_body(cl, carry):
        i = wid * CPW + cl
        sel = lane == (cl % 16)
        in_hi = cl >= 16
        xi = _splat(xc, sel, in_hi)
        yi = _splat(yc, sel, in_hi)
        zi = _splat(zc, sel, in_hi)

        def chunk_body(cj, n):
            base = cj * 16
            xj = xv[pl.ds(base, 16)]
            yj = yv[pl.ds(base, 16)]
            zj = zv[pl.ds(base, 16)]
            dx, dy, dz = xi - xj, yi - yj, zi - zj
            sxf = jnp.where(dx >= half, one, zero) - jnp.where(dx <= -half, one, zero)
            syf = jnp.where(dy >= half, one, zero) - jnp.where(dy <= -half, one, zero)
            szf = jnp.where(dz >= half, one, zero) - jnp.where(dz <= -half, one, zero)
            cx = (sxf + xj) - xi
            cy = (syf + yj) - yi
            cz = (szf + zj) - zi
            dist2 = (cx * cx + cy * cy) + cz * cz
            jvec = base + lane
            m = (dist2 <= THRESH) & (jvec != i)
            cell = ((szf.astype(jnp.int32) + 1) * 9
                    + (syf.astype(jnp.int32) + 1) * 3
                    + (sxf.astype(jnp.int32) + 1))
            key = cell * N_POINTS + jvec
            plsc.store_compressed(buf.at[pl.ds(n, 16)], key, mask=m)
            pc = plsc.all_reduce_population_count(m)
            return n + jnp.max(pc)

        n = lax.fori_loop(0, N_POINTS // 16, chunk_body, jnp.int32(0))
        for k in range(4):
            buf[pl.ds(n + k * 16, 16)] = largev

        def load_block(b):
            o = b * MAXN
            return tuple(buf[pl.ds(o + k * 16, 16)] for k in range(4))

        acc = _sort64(*load_block(0))

        def mcond(st):
            return st[0] * MAXN < n

        def mbody(st):
            b = st[0]
            blk = _sort64(*load_block(b))
            return (b + 1,) + _merge_low64(st[1:], blk)

        st = lax.while_loop(mcond, mbody, (jnp.int32(1),) + acc)
        for k in range(4):
            kstage[cl, pl.ds(k * 16, 16)] = st[1 + k]
        plsc.store_scatter(cstage, [jnp.full((16,), 1, jnp.int32) * cl],
                           jnp.full((16,), 1, jnp.int32) * n, mask=lane == 0)
        return carry

    lax.fori_loop(0, CPW, center_body, jnp.int32(0))
    pltpu.sync_copy(kstage, keys_hbm.at[pl.ds(wid * CPW, CPW)])
    pltpu.sync_copy(cstage, counts_hbm.at[pl.ds(wid * CPW, CPW)])


def kernel(positions, max_neighbours):
    x = positions[:, 0]
    y = positions[:, 1]
    z = positions[:, 2]
    keys, counts = _neighbour_kernel(x, y, z)
    to_idx = jnp.where(keys < N_CELLS * N_POINTS, keys, -1)
    to_idx = jnp.where(jnp.arange(MAXN) < max_neighbours, to_idx, -1)
    neighbours = jnp.where(to_idx < 0, -1, to_idx % N_POINTS)
    cell = jnp.where(to_idx < 0, N_CELLS - 1, to_idx // N_POINTS)
    cell_indices = jnp.stack(
        [cell % 3 - 1, (cell // 3) % 3 - 1, cell // 9 - 1], axis=-1
    ).astype(jnp.int32)
    actual_max_neighbours = jnp.max(counts)
    return neighbours, cell_indices, actual_max_neighbours
